# Initial kernel scaffold; baseline (speedup 1.0000x reference)
import functools

import jax
import jax.numpy as jnp
from jax import lax
from jax.experimental import pallas as pl
from jax.experimental.pallas import tpu as pltpu
from jax.experimental.pallas import tpu_sc as plsc

N_NODES = 50000
EMB = 64
N_EDGES = 800000

NC = 2
NS = 16
CHUNK = 128
N_CHUNKS = 391
EDGES_PER_SUB = CHUNK * N_CHUNKS
EP = EDGES_PER_SUB * NS
HALF = N_NODES // NC
ZSTRIPE = 1568
ACC_ROWS = ZSTRIPE * NS
TRASH = ACC_ROWS - 8
WLAST = HALF - 15 * ZSTRIPE


def _spmm_body(h_hbm, col_hbm, row_hbm, val_hbm, zero_hbm, out_hbm,
               acc, col_v, row_v, val_v, idx_v, rows_v, sem):
    c = lax.axis_index("c")
    s = lax.axis_index("s")
    base_row = c * HALF

    pltpu.sync_copy(zero_hbm, acc.at[pl.ds(s * ZSTRIPE, ZSTRIPE)])
    plsc.subcore_barrier()

    ebase = s * EDGES_PER_SUB

    def chunk_body(g, carry):
        off = ebase + g * CHUNK
        pltpu.sync_copy(col_hbm.at[pl.ds(off, CHUNK)], col_v)
        pltpu.sync_copy(row_hbm.at[pl.ds(off, CHUNK)], row_v)
        pltpu.sync_copy(val_hbm.at[pl.ds(off, CHUNK)], val_v)
        pltpu.async_copy(h_hbm.at[col_v], rows_v, sem).wait()

        for q in range(CHUNK // 16):
            r16 = row_v[pl.ds(q * 16, 16)]
            loc = r16 - base_row
            ok = (loc >= 0) & (loc < HALF)
            idx_v[pl.ds(q * 16, 16)] = jnp.where(ok, loc, TRASH)

        def q_body(q, carry2):
            e0 = q * 16
            for e in range(16):
                vb = plsc.load_gather(
                    val_v, [jnp.full((16,), e0 + e, jnp.int32)])
                for j in range(EMB // 16):
                    sl = (e0 + e, pl.ds(j * 16, 16))
                    rows_v[sl] = rows_v[sl] * vb
            return carry2

        lax.fori_loop(0, CHUNK // 16, q_body, 0)

        pltpu.sync_copy(rows_v, acc.at[idx_v], add=True)
        return carry

    lax.fori_loop(0, N_CHUNKS, chunk_body, 0)
    plsc.subcore_barrier()

    @pl.when(s < NS - 1)
    def _():
        pltpu.sync_copy(acc.at[pl.ds(s * ZSTRIPE, ZSTRIPE)],
                        out_hbm.at[pl.ds(base_row + s * ZSTRIPE, ZSTRIPE)])

    @pl.when(s == NS - 1)
    def _():
        pltpu.sync_copy(acc.at[pl.ds((NS - 1) * ZSTRIPE, WLAST)],
                        out_hbm.at[pl.ds(base_row + (NS - 1) * ZSTRIPE, WLAST)])


_spmm = pl.kernel(
    _spmm_body,
    out_type=jax.ShapeDtypeStruct((N_NODES, EMB), jnp.float32),
    mesh=plsc.VectorSubcoreMesh(core_axis_name="c", subcore_axis_name="s"),
    scratch_types=[
        pltpu.VMEM_SHARED((ACC_ROWS, EMB), jnp.float32),
        pltpu.VMEM((CHUNK,), jnp.int32),
        pltpu.VMEM((CHUNK,), jnp.int32),
        pltpu.VMEM((CHUNK,), jnp.float32),
        pltpu.VMEM((CHUNK,), jnp.int32),
        pltpu.VMEM((CHUNK, EMB), jnp.float32),
        pltpu.SemaphoreType.DMA,
    ],
)


def _mm_body(x_ref, w_ref, o_ref):
    o_ref[...] = jnp.dot(x_ref[...], w_ref[...],
                         preferred_element_type=jnp.float32)


def _tc_mm(x, wt, blk):
    n = x.shape[0]
    return pl.pallas_call(
        _mm_body,
        grid=(n // blk,),
        in_specs=[pl.BlockSpec((blk, EMB), lambda i: (i, 0)),
                  pl.BlockSpec((EMB, EMB), lambda i: (0, 0))],
        out_specs=pl.BlockSpec((blk, EMB), lambda i: (i, 0)),
        out_shape=jax.ShapeDtypeStruct((n, EMB), jnp.float32),
    )(x, wt)


def _act_mm_body(s_ref, b_ref, w_ref, ego_ref, h_ref):
    y = s_ref[...] + b_ref[...]
    ego = jnp.where(y >= 0, y, 0.01 * y)
    ego_ref[...] = ego
    h_ref[...] = jnp.dot(ego, w_ref[...], preferred_element_type=jnp.float32)


def _tc_act_mm(s, bias, wt, blk):
    n = s.shape[0]
    return pl.pallas_call(
        _act_mm_body,
        grid=(n // blk,),
        in_specs=[pl.BlockSpec((blk, EMB), lambda i: (i, 0)),
                  pl.BlockSpec((1, EMB), lambda i: (0, 0)),
                  pl.BlockSpec((EMB, EMB), lambda i: (0, 0))],
        out_specs=[pl.BlockSpec((blk, EMB), lambda i: (i, 0)),
                   pl.BlockSpec((blk, EMB), lambda i: (i, 0))],
        out_shape=[jax.ShapeDtypeStruct((n, EMB), jnp.float32),
                   jax.ShapeDtypeStruct((n, EMB), jnp.float32)],
    )(s, bias, wt)


def _final_body(s_ref, b_ref, emb_ref, ego1_ref, o_ref):
    y = s_ref[...] + b_ref[...]
    ego2 = jnp.where(y >= 0, y, 0.01 * y)
    o_ref[...] = (emb_ref[...] + ego1_ref[...] + ego2) * (1.0 / 3.0)


def _tc_final(s, bias, emb, ego1, blk):
    n = s.shape[0]
    return pl.pallas_call(
        _final_body,
        grid=(n // blk,),
        in_specs=[pl.BlockSpec((blk, EMB), lambda i: (i, 0)),
                  pl.BlockSpec((1, EMB), lambda i: (0, 0)),
                  pl.BlockSpec((blk, EMB), lambda i: (i, 0)),
                  pl.BlockSpec((blk, EMB), lambda i: (i, 0))],
        out_specs=pl.BlockSpec((blk, EMB), lambda i: (i, 0)),
        out_shape=jax.ShapeDtypeStruct((n, EMB), jnp.float32),
    )(s, bias, emb, ego1)


def kernel(adj_index, adj_values, emb_weight, W, b):
    row = adj_index[0]
    col = adj_index[1]
    pad = EP - N_EDGES
    col_p = jnp.concatenate([col, jnp.zeros((pad,), jnp.int32)])
    row_p = jnp.concatenate([row, jnp.full((pad,), 1 << 29, jnp.int32)])
    val_p = jnp.concatenate([adj_values, jnp.zeros((pad,), jnp.float32)])
    zeros_blk = jnp.zeros((ZSTRIPE, EMB), jnp.float32)

    h0 = _tc_mm(emb_weight, W[0].T, 5000)
    s0 = _spmm(h0, col_p, row_p, val_p, zeros_blk)
    ego1, h1 = _tc_act_mm(s0, b[0].reshape(1, EMB), W[1].T, 5000)
    s1 = _spmm(h1, col_p, row_p, val_p, zeros_blk)
    out = _tc_final(s1[:30000], b[1].reshape(1, EMB),
                    emb_weight[:30000], ego1[:30000], 3000)
    return out[:15000], out[15000:30000]


# speedup vs baseline: 2.0185x; 2.0185x over previous
import functools

import jax
import jax.numpy as jnp
from jax import lax
from jax.experimental import pallas as pl
from jax.experimental.pallas import tpu as pltpu
from jax.experimental.pallas import tpu_sc as plsc

N_NODES = 50000
EMB = 64
N_EDGES = 800000

NC = 2
NS = 16
CHUNK = 128
N_CHUNKS = 391
EDGES_PER_SUB = CHUNK * N_CHUNKS
EP = EDGES_PER_SUB * NS
HALF = N_NODES // NC
ZSTRIPE = 1568
ACC_ROWS = ZSTRIPE * NS
TRASH = ACC_ROWS - 8
WLAST = HALF - 15 * ZSTRIPE


def _spmm_body(h_hbm, col_hbm, row_hbm, val_hbm, zero_hbm, out_hbm,
               acc, col_v, row_v, val_v, idx_v, rows_v, sem):
    c = lax.axis_index("c")
    s = lax.axis_index("s")
    base_row = c * HALF

    pltpu.sync_copy(zero_hbm, acc.at[pl.ds(s * ZSTRIPE, ZSTRIPE)])
    plsc.subcore_barrier()

    ebase = s * EDGES_PER_SUB

    def chunk_body(g, carry):
        off = ebase + g * CHUNK
        pltpu.sync_copy(col_hbm.at[pl.ds(off, CHUNK)], col_v)
        pltpu.sync_copy(row_hbm.at[pl.ds(off, CHUNK)], row_v)
        pltpu.sync_copy(val_hbm.at[pl.ds(off, CHUNK)], val_v)
        pltpu.async_copy(h_hbm.at[col_v], rows_v, sem).wait()

        for q in range(CHUNK // 16):
            r16 = row_v[pl.ds(q * 16, 16)]
            loc = r16 - base_row
            ok = (loc >= 0) & (loc < HALF)
            idx_v[pl.ds(q * 16, 16)] = jnp.where(ok, loc, TRASH)

        def q_body(q, carry2):
            e0 = q * 16
            val16 = val_v[pl.ds(e0, 16)]
            for e in range(16):
                vb = val16[e]
                for j in range(EMB // 16):
                    sl = (e0 + e, pl.ds(j * 16, 16))
                    rows_v[sl] = rows_v[sl] * vb
            return carry2

        lax.fori_loop(0, CHUNK // 16, q_body, 0)

        pltpu.sync_copy(rows_v, acc.at[idx_v], add=True)
        return carry

    lax.fori_loop(0, N_CHUNKS, chunk_body, 0)
    plsc.subcore_barrier()

    @pl.when(s < NS - 1)
    def _():
        pltpu.sync_copy(acc.at[pl.ds(s * ZSTRIPE, ZSTRIPE)],
                        out_hbm.at[pl.ds(base_row + s * ZSTRIPE, ZSTRIPE)])

    @pl.when(s == NS - 1)
    def _():
        pltpu.sync_copy(acc.at[pl.ds((NS - 1) * ZSTRIPE, WLAST)],
                        out_hbm.at[pl.ds(base_row + (NS - 1) * ZSTRIPE, WLAST)])


_spmm = pl.kernel(
    _spmm_body,
    out_type=jax.ShapeDtypeStruct((N_NODES, EMB), jnp.float32),
    mesh=plsc.VectorSubcoreMesh(core_axis_name="c", subcore_axis_name="s"),
    compiler_params=pltpu.CompilerParams(use_tc_tiling_on_sc=False),
    scratch_types=[
        pltpu.VMEM_SHARED((ACC_ROWS, EMB), jnp.float32),
        pltpu.VMEM((CHUNK,), jnp.int32),
        pltpu.VMEM((CHUNK,), jnp.int32),
        pltpu.VMEM((CHUNK,), jnp.float32),
        pltpu.VMEM((CHUNK,), jnp.int32),
        pltpu.VMEM((CHUNK, EMB), jnp.float32),
        pltpu.SemaphoreType.DMA,
    ],
)


def _mm_body(x_ref, w_ref, o_ref):
    o_ref[...] = jnp.dot(x_ref[...], w_ref[...],
                         preferred_element_type=jnp.float32)


def _tc_mm(x, wt, blk):
    n = x.shape[0]
    return pl.pallas_call(
        _mm_body,
        grid=(n // blk,),
        in_specs=[pl.BlockSpec((blk, EMB), lambda i: (i, 0)),
                  pl.BlockSpec((EMB, EMB), lambda i: (0, 0))],
        out_specs=pl.BlockSpec((blk, EMB), lambda i: (i, 0)),
        out_shape=jax.ShapeDtypeStruct((n, EMB), jnp.float32),
    )(x, wt)


def _act_mm_body(s_ref, b_ref, w_ref, ego_ref, h_ref):
    y = s_ref[...] + b_ref[...]
    ego = jnp.where(y >= 0, y, 0.01 * y)
    ego_ref[...] = ego
    h_ref[...] = jnp.dot(ego, w_ref[...], preferred_element_type=jnp.float32)


def _tc_act_mm(s, bias, wt, blk):
    n = s.shape[0]
    return pl.pallas_call(
        _act_mm_body,
        grid=(n // blk,),
        in_specs=[pl.BlockSpec((blk, EMB), lambda i: (i, 0)),
                  pl.BlockSpec((1, EMB), lambda i: (0, 0)),
                  pl.BlockSpec((EMB, EMB), lambda i: (0, 0))],
        out_specs=[pl.BlockSpec((blk, EMB), lambda i: (i, 0)),
                   pl.BlockSpec((blk, EMB), lambda i: (i, 0))],
        out_shape=[jax.ShapeDtypeStruct((n, EMB), jnp.float32),
                   jax.ShapeDtypeStruct((n, EMB), jnp.float32)],
    )(s, bias, wt)


def _final_body(s_ref, b_ref, emb_ref, ego1_ref, o_ref):
    y = s_ref[...] + b_ref[...]
    ego2 = jnp.where(y >= 0, y, 0.01 * y)
    o_ref[...] = (emb_ref[...] + ego1_ref[...] + ego2) * (1.0 / 3.0)


def _tc_final(s, bias, emb, ego1, blk):
    n = s.shape[0]
    return pl.pallas_call(
        _final_body,
        grid=(n // blk,),
        in_specs=[pl.BlockSpec((blk, EMB), lambda i: (i, 0)),
                  pl.BlockSpec((1, EMB), lambda i: (0, 0)),
                  pl.BlockSpec((blk, EMB), lambda i: (i, 0)),
                  pl.BlockSpec((blk, EMB), lambda i: (i, 0))],
        out_specs=pl.BlockSpec((blk, EMB), lambda i: (i, 0)),
        out_shape=jax.ShapeDtypeStruct((n, EMB), jnp.float32),
    )(s, bias, emb, ego1)


def kernel(adj_index, adj_values, emb_weight, W, b):
    row = adj_index[0]
    col = adj_index[1]
    pad = EP - N_EDGES
    col_p = jnp.concatenate([col, jnp.zeros((pad,), jnp.int32)])
    row_p = jnp.concatenate([row, jnp.full((pad,), 1 << 29, jnp.int32)])
    val_p = jnp.concatenate([adj_values, jnp.zeros((pad,), jnp.float32)])
    zeros_blk = jnp.zeros((ZSTRIPE, EMB), jnp.float32)

    h0 = _tc_mm(emb_weight, W[0].T, 5000)
    s0 = _spmm(h0, col_p, row_p, val_p, zeros_blk)
    ego1, h1 = _tc_act_mm(s0, b[0].reshape(1, EMB), W[1].T, 5000)
    s1 = _spmm(h1, col_p, row_p, val_p, zeros_blk)
    out = _tc_final(s1[:30000], b[1].reshape(1, EMB),
                    emb_weight[:30000], ego1[:30000], 3000)
    return out[:15000], out[15000:30000]


# speedup vs baseline: 2.7505x; 1.3626x over previous
import functools

import jax
import jax.numpy as jnp
from jax import lax
from jax.experimental import pallas as pl
from jax.experimental.pallas import tpu as pltpu
from jax.experimental.pallas import tpu_sc as plsc

N_NODES = 50000
EMB = 64
N_EDGES = 800000

NC = 2
NS = 16
CHUNK = 128
SUP = 128
SS = 400
EDGES_PER_SUB = SUP * SS
EP = EDGES_PER_SUB * NS
NSUPER = SS * NS
HALF = N_NODES // NC
ZSTRIPE = 1568
ACC_ROWS = ZSTRIPE * NS
TRASH0 = 25008
WLAST = HALF - 15 * ZSTRIPE


def _spmm_body(h_hbm, packed_hbm, vals_hbm, zero_hbm, out_hbm,
               acc, pkt4, val4, rows_v, idx_v,
               psem0, psem1, psem2, psem3, gsem0, gsem1, ssem0, ssem1):
    c = lax.axis_index("c")
    s = lax.axis_index("s")
    base_row = c * HALF
    trash = TRASH0 + s

    pltpu.sync_copy(zero_hbm, acc.at[pl.ds(s * ZSTRIPE, ZSTRIPE)])
    plsc.subcore_barrier()

    psem = [psem0, psem1, psem2, psem3]
    gsem = [gsem0, gsem1]
    ssem = [ssem0, ssem1]
    sbase = s * SS

    def issue_pkt(slot, t):
        pltpu.async_copy(packed_hbm.at[sbase + t], pkt4.at[slot], psem[slot])
        pltpu.async_copy(vals_hbm.at[sbase + t], val4.at[slot], psem[slot])

    def wait_pkt(slot):
        pltpu.make_async_copy(packed_hbm.at[0], pkt4.at[slot],
                              psem[slot]).wait()
        pltpu.make_async_copy(vals_hbm.at[0], val4.at[slot],
                              psem[slot]).wait()

    def issue_gathers(p, slot):
        for j in range(SUP // CHUNK):
            pltpu.async_copy(
                h_hbm.at[pkt4.at[slot, 0, pl.ds(j * CHUNK, CHUNK)]],
                rows_v.at[p, pl.ds(j * CHUNK, CHUNK)], gsem[p])

    def wait_gathers(p):
        for j in range(SUP // CHUNK):
            pltpu.make_async_copy(h_hbm.at[pl.ds(0, CHUNK)],
                                  rows_v.at[p, pl.ds(j * CHUNK, CHUNK)],
                                  gsem[p]).wait()

    def issue_scatters(p):
        for j in range(SUP // CHUNK):
            pltpu.async_copy(rows_v.at[p, pl.ds(j * CHUNK, CHUNK)],
                             acc.at[idx_v.at[p, j]], ssem[p], add=True)

    def wait_scatters(p):
        for j in range(SUP // CHUNK):
            pltpu.make_async_copy(h_hbm.at[pl.ds(0, CHUNK)],
                                  acc.at[pl.ds(j * CHUNK, CHUNK)],
                                  ssem[p]).wait()

    def compute(k, p):
        def q_body(q, carry):
            e0 = q * 16
            r16 = pkt4[k, 1, pl.ds(e0, 16)]
            loc = r16 - base_row
            ok = (loc >= 0) & (loc < HALF)
            idx_v[p, q // 8, pl.ds((q % 8) * 16, 16)] = jnp.where(
                ok, loc, trash)
            v16 = val4[k, pl.ds(e0, 16)]
            for e in range(16):
                vb = v16[e]
                for j in range(EMB // 16):
                    sl = (p, e0 + e, pl.ds(j * 16, 16))
                    rows_v[sl] = rows_v[sl] * vb
            return carry

        lax.fori_loop(0, SUP // 16, q_body, 0)

    issue_pkt(0, 0)
    issue_pkt(1, 1)
    wait_pkt(0)
    issue_gathers(0, 0)

    def tt_body(tt, carry):
        for k in range(4):
            t = tt * 4 + k
            p = k & 1
            q = 1 - p
            wait_gathers(p)

            @pl.when(t + 2 < SS)
            def _():
                issue_pkt((k + 2) & 3, t + 2)

            @pl.when(t + 1 < SS)
            def _():
                wait_pkt((k + 1) & 3)

                @pl.when(t >= 1)
                def _():
                    wait_scatters(q)

                issue_gathers(q, (k + 1) & 3)

            compute(k, p)
            issue_scatters(p)
        return carry

    lax.fori_loop(0, SS // 4, tt_body, 0)
    wait_scatters(0)
    wait_scatters(1)
    plsc.subcore_barrier()

    @pl.when(s < NS - 1)
    def _():
        pltpu.sync_copy(acc.at[pl.ds(s * ZSTRIPE, ZSTRIPE)],
                        out_hbm.at[pl.ds(base_row + s * ZSTRIPE, ZSTRIPE)])

    @pl.when(s == NS - 1)
    def _():
        pltpu.sync_copy(acc.at[pl.ds((NS - 1) * ZSTRIPE, WLAST)],
                        out_hbm.at[pl.ds(base_row + (NS - 1) * ZSTRIPE, WLAST)])


_spmm = pl.kernel(
    _spmm_body,
    out_type=jax.ShapeDtypeStruct((N_NODES, EMB), jnp.float32),
    mesh=plsc.VectorSubcoreMesh(core_axis_name="c", subcore_axis_name="s"),
    compiler_params=pltpu.CompilerParams(use_tc_tiling_on_sc=False),
    scratch_types=[
        pltpu.VMEM_SHARED((ACC_ROWS, EMB), jnp.float32),
        pltpu.VMEM((4, 2, SUP), jnp.int32),
        pltpu.VMEM((4, SUP), jnp.float32),
        pltpu.VMEM((2, SUP, EMB), jnp.float32),
        pltpu.VMEM((2, SUP // CHUNK, CHUNK), jnp.int32),
        pltpu.SemaphoreType.DMA,
        pltpu.SemaphoreType.DMA,
        pltpu.SemaphoreType.DMA,
        pltpu.SemaphoreType.DMA,
        pltpu.SemaphoreType.DMA,
        pltpu.SemaphoreType.DMA,
        pltpu.SemaphoreType.DMA,
        pltpu.SemaphoreType.DMA,
    ],
)


def _mm_body(x_ref, w_ref, o_ref):
    o_ref[...] = jnp.dot(x_ref[...], w_ref[...],
                         preferred_element_type=jnp.float32)


def _tc_mm(x, wt, blk):
    n = x.shape[0]
    return pl.pallas_call(
        _mm_body,
        grid=(n // blk,),
        in_specs=[pl.BlockSpec((blk, EMB), lambda i: (i, 0)),
                  pl.BlockSpec((EMB, EMB), lambda i: (0, 0))],
        out_specs=pl.BlockSpec((blk, EMB), lambda i: (i, 0)),
        out_shape=jax.ShapeDtypeStruct((n, EMB), jnp.float32),
    )(x, wt)


def _act_mm_body(s_ref, b_ref, w_ref, ego_ref, h_ref):
    y = s_ref[...] + b_ref[...]
    ego = jnp.where(y >= 0, y, 0.01 * y)
    ego_ref[...] = ego
    h_ref[...] = jnp.dot(ego, w_ref[...], preferred_element_type=jnp.float32)


def _tc_act_mm(s, bias, wt, blk):
    n = s.shape[0]
    return pl.pallas_call(
        _act_mm_body,
        grid=(n // blk,),
        in_specs=[pl.BlockSpec((blk, EMB), lambda i: (i, 0)),
                  pl.BlockSpec((1, EMB), lambda i: (0, 0)),
                  pl.BlockSpec((EMB, EMB), lambda i: (0, 0))],
        out_specs=[pl.BlockSpec((blk, EMB), lambda i: (i, 0)),
                   pl.BlockSpec((blk, EMB), lambda i: (i, 0))],
        out_shape=[jax.ShapeDtypeStruct((n, EMB), jnp.float32),
                   jax.ShapeDtypeStruct((n, EMB), jnp.float32)],
    )(s, bias, wt)


def _final_body(s_ref, b_ref, emb_ref, ego1_ref, o_ref):
    y = s_ref[...] + b_ref[...]
    ego2 = jnp.where(y >= 0, y, 0.01 * y)
    o_ref[...] = (emb_ref[...] + ego1_ref[...] + ego2) * (1.0 / 3.0)


def _tc_final(s, bias, emb, ego1, blk):
    n = s.shape[0]
    return pl.pallas_call(
        _final_body,
        grid=(n // blk,),
        in_specs=[pl.BlockSpec((blk, EMB), lambda i: (i, 0)),
                  pl.BlockSpec((1, EMB), lambda i: (0, 0)),
                  pl.BlockSpec((blk, EMB), lambda i: (i, 0)),
                  pl.BlockSpec((blk, EMB), lambda i: (i, 0))],
        out_specs=pl.BlockSpec((blk, EMB), lambda i: (i, 0)),
        out_shape=jax.ShapeDtypeStruct((n, EMB), jnp.float32),
    )(s, bias, emb, ego1)


def kernel(adj_index, adj_values, emb_weight, W, b):
    row = adj_index[0]
    col = adj_index[1]
    pad = EP - N_EDGES
    col_p = jnp.concatenate([col, jnp.zeros((pad,), jnp.int32)])
    row_p = jnp.concatenate([row, jnp.full((pad,), 1 << 29, jnp.int32)])
    val_p = jnp.concatenate([adj_values, jnp.zeros((pad,), jnp.float32)])
    packed = (jnp.stack([col_p, row_p], axis=0)
              .reshape(2, NSUPER, SUP).transpose(1, 0, 2))
    vals = val_p.reshape(NSUPER, SUP)
    zeros_blk = jnp.zeros((ZSTRIPE, EMB), jnp.float32)

    h0 = _tc_mm(emb_weight, W[0].T, 5000)
    s0 = _spmm(h0, packed, vals, zeros_blk)
    ego1, h1 = _tc_act_mm(s0, b[0].reshape(1, EMB), W[1].T, 5000)
    s1 = _spmm(h1, packed, vals, zeros_blk)
    out = _tc_final(s1[:30000], b[1].reshape(1, EMB),
                    emb_weight[:30000], ego1[:30000], 3000)
    return out[:15000], out[15000:30000]


# speedup vs baseline: 5.1711x; 1.8801x over previous
import functools

import jax
import jax.numpy as jnp
from jax import lax
from jax.experimental import pallas as pl
from jax.experimental.pallas import tpu as pltpu
from jax.experimental.pallas import tpu_sc as plsc

N_NODES = 50000
EMB = 64
N_EDGES = 800000

NC = 2
NS = 16
CHUNK = 128
SUP = 128
SS = 402
EDGES_PER_SUB = SUP * SS
EP = EDGES_PER_SUB * NS
NSUPER = SS * NS
HALF = N_NODES // NC
ZSTRIPE = 1564
ACC_ROWS = ZSTRIPE * NS
TRASH0 = 25008
WLAST = HALF - 15 * ZSTRIPE


def _spmm_body(h_hbm, packed_hbm, vals_hbm, zero_hbm, out_hbm,
               acc, pkt6, val6, rows_v, idx_v,
               psem0, psem1, psem2, psem3, psem4, psem5,
               gsem0, gsem1, gsem2, ssem0, ssem1, ssem2):
    c = lax.axis_index("c")
    s = lax.axis_index("s")
    base_row = c * HALF
    trash = TRASH0 + s

    pltpu.sync_copy(zero_hbm, acc.at[pl.ds(s * ZSTRIPE, ZSTRIPE)])
    plsc.subcore_barrier()

    psem = [psem0, psem1, psem2, psem3, psem4, psem5]
    gsem = [gsem0, gsem1, gsem2]
    ssem = [ssem0, ssem1, ssem2]
    sbase = s * SS

    def issue_pkt(slot, t):
        pltpu.async_copy(packed_hbm.at[sbase + t], pkt6.at[slot], psem[slot])
        pltpu.async_copy(vals_hbm.at[sbase + t], val6.at[slot], psem[slot])

    def wait_pkt(slot):
        pltpu.make_async_copy(packed_hbm.at[0], pkt6.at[slot],
                              psem[slot]).wait()
        pltpu.make_async_copy(vals_hbm.at[0], val6.at[slot],
                              psem[slot]).wait()

    def issue_gathers(p, slot):
        pltpu.async_copy(h_hbm.at[pkt6.at[slot, 0]], rows_v.at[p], gsem[p])

    def wait_gathers(p):
        pltpu.make_async_copy(h_hbm.at[pl.ds(0, SUP)], rows_v.at[p],
                              gsem[p]).wait()

    def issue_scatters(p):
        pltpu.async_copy(rows_v.at[p], acc.at[idx_v.at[p]], ssem[p], add=True)

    def wait_scatters(p):
        pltpu.make_async_copy(h_hbm.at[pl.ds(0, SUP)], acc.at[pl.ds(0, SUP)],
                              ssem[p]).wait()

    def compute(k, p):
        def q_body(q, carry):
            e0 = q * 16
            r16 = pkt6[k, 1, pl.ds(e0, 16)]
            loc = r16 - base_row
            ok = (loc >= 0) & (loc < HALF)
            idx_v[p, pl.ds(e0, 16)] = jnp.where(ok, loc, trash)
            v16 = val6[k, pl.ds(e0, 16)]
            for e in range(16):
                vb = v16.at[jnp.full((16,), e, jnp.int32)].get(
                    mode="promise_in_bounds")
                for j in range(EMB // 16):
                    sl = (p, e0 + e, pl.ds(j * 16, 16))
                    rows_v[sl] = rows_v[sl] * vb
            return carry

        lax.fori_loop(0, SUP // 16, q_body, 0)

    issue_pkt(0, 0)
    issue_pkt(1, 1)
    issue_pkt(2, 2)
    wait_pkt(0)
    issue_gathers(0, 0)
    wait_pkt(1)
    issue_gathers(1, 1)

    def tt_body(tt, carry):
        for k in range(6):
            t = tt * 6 + k
            p = k % 3

            wait_gathers(p)

            @pl.when(t + 3 < SS)
            def _():
                issue_pkt((k + 3) % 6, t + 3)

            @pl.when(t + 2 < SS)
            def _():
                wait_pkt((k + 2) % 6)

                @pl.when(t >= 1)
                def _():
                    wait_scatters((k + 2) % 3)

                issue_gathers((k + 2) % 3, (k + 2) % 6)

            compute(k, p)
            issue_scatters(p)
        return carry

    lax.fori_loop(0, SS // 6, tt_body, 0)
    wait_scatters(0)
    wait_scatters(1)
    wait_scatters(2)
    plsc.subcore_barrier()

    @pl.when(s < NS - 1)
    def _():
        pltpu.sync_copy(acc.at[pl.ds(s * ZSTRIPE, ZSTRIPE)],
                        out_hbm.at[pl.ds(base_row + s * ZSTRIPE, ZSTRIPE)])

    @pl.when(s == NS - 1)
    def _():
        pltpu.sync_copy(acc.at[pl.ds((NS - 1) * ZSTRIPE, WLAST)],
                        out_hbm.at[pl.ds(base_row + (NS - 1) * ZSTRIPE, WLAST)])


_spmm = pl.kernel(
    _spmm_body,
    out_type=jax.ShapeDtypeStruct((N_NODES, EMB), jnp.float32),
    mesh=plsc.VectorSubcoreMesh(core_axis_name="c", subcore_axis_name="s"),
    compiler_params=pltpu.CompilerParams(use_tc_tiling_on_sc=False),
    scratch_types=[
        pltpu.VMEM_SHARED((ACC_ROWS, EMB), jnp.float32),
        pltpu.VMEM((6, 2, SUP), jnp.int32),
        pltpu.VMEM((6, SUP), jnp.float32),
        pltpu.VMEM((3, SUP, EMB), jnp.float32),
        pltpu.VMEM((3, SUP), jnp.int32),
        pltpu.SemaphoreType.DMA,
        pltpu.SemaphoreType.DMA,
        pltpu.SemaphoreType.DMA,
        pltpu.SemaphoreType.DMA,
        pltpu.SemaphoreType.DMA,
        pltpu.SemaphoreType.DMA,
        pltpu.SemaphoreType.DMA,
        pltpu.SemaphoreType.DMA,
        pltpu.SemaphoreType.DMA,
        pltpu.SemaphoreType.DMA,
        pltpu.SemaphoreType.DMA,
        pltpu.SemaphoreType.DMA,
    ],
)


def _mm_body(x_ref, w_ref, o_ref):
    o_ref[...] = jnp.dot(x_ref[...], w_ref[...],
                         preferred_element_type=jnp.float32)


def _tc_mm(x, wt, blk):
    n = x.shape[0]
    return pl.pallas_call(
        _mm_body,
        grid=(n // blk,),
        in_specs=[pl.BlockSpec((blk, EMB), lambda i: (i, 0)),
                  pl.BlockSpec((EMB, EMB), lambda i: (0, 0))],
        out_specs=pl.BlockSpec((blk, EMB), lambda i: (i, 0)),
        out_shape=jax.ShapeDtypeStruct((n, EMB), jnp.float32),
    )(x, wt)


def _act_mm_body(s_ref, b_ref, w_ref, ego_ref, h_ref):
    y = s_ref[...] + b_ref[...]
    ego = jnp.where(y >= 0, y, 0.01 * y)
    ego_ref[...] = ego
    h_ref[...] = jnp.dot(ego, w_ref[...], preferred_element_type=jnp.float32)


def _tc_act_mm(s, bias, wt, blk):
    n = s.shape[0]
    return pl.pallas_call(
        _act_mm_body,
        grid=(n // blk,),
        in_specs=[pl.BlockSpec((blk, EMB), lambda i: (i, 0)),
                  pl.BlockSpec((1, EMB), lambda i: (0, 0)),
                  pl.BlockSpec((EMB, EMB), lambda i: (0, 0))],
        out_specs=[pl.BlockSpec((blk, EMB), lambda i: (i, 0)),
                   pl.BlockSpec((blk, EMB), lambda i: (i, 0))],
        out_shape=[jax.ShapeDtypeStruct((n, EMB), jnp.float32),
                   jax.ShapeDtypeStruct((n, EMB), jnp.float32)],
    )(s, bias, wt)


def _final_body(s_ref, b_ref, emb_ref, ego1_ref, o_ref):
    y = s_ref[...] + b_ref[...]
    ego2 = jnp.where(y >= 0, y, 0.01 * y)
    o_ref[...] = (emb_ref[...] + ego1_ref[...] + ego2) * (1.0 / 3.0)


def _tc_final(s, bias, emb, ego1, blk):
    n = s.shape[0]
    return pl.pallas_call(
        _final_body,
        grid=(n // blk,),
        in_specs=[pl.BlockSpec((blk, EMB), lambda i: (i, 0)),
                  pl.BlockSpec((1, EMB), lambda i: (0, 0)),
                  pl.BlockSpec((blk, EMB), lambda i: (i, 0)),
                  pl.BlockSpec((blk, EMB), lambda i: (i, 0))],
        out_specs=pl.BlockSpec((blk, EMB), lambda i: (i, 0)),
        out_shape=jax.ShapeDtypeStruct((n, EMB), jnp.float32),
    )(s, bias, emb, ego1)


def kernel(adj_index, adj_values, emb_weight, W, b):
    row = adj_index[0]
    col = adj_index[1]
    pad = EP - N_EDGES
    pad_cols = (jnp.arange(pad, dtype=jnp.int32) * 37) % N_NODES
    col_p = jnp.concatenate([col, pad_cols])
    row_p = jnp.concatenate([row, jnp.full((pad,), 1 << 29, jnp.int32)])
    val_p = jnp.concatenate([adj_values, jnp.zeros((pad,), jnp.float32)])
    packed = (jnp.stack([col_p, row_p], axis=0)
              .reshape(2, NSUPER, SUP).transpose(1, 0, 2))
    vals = val_p.reshape(NSUPER, SUP)
    zeros_blk = jnp.zeros((ZSTRIPE, EMB), jnp.float32)

    h0 = _tc_mm(emb_weight, W[0].T, 5000)
    s0 = _spmm(h0, packed, vals, zeros_blk)
    ego1, h1 = _tc_act_mm(s0, b[0].reshape(1, EMB), W[1].T, 5000)
    s1 = _spmm(h1, packed, vals, zeros_blk)
    out = _tc_final(s1[:30000], b[1].reshape(1, EMB),
                    emb_weight[:30000], ego1[:30000], 3000)
    return out[:15000], out[15000:30000]
